# baseline (speedup 1.0000x reference)
import jax
import jax.numpy as jnp
from jax import lax
from jax.experimental import pallas as pl
from jax.experimental.pallas import tpu as pltpu
from jax.experimental.pallas import tpu_sc as plsc

BATCH = 4
SEQ_LEN = 8192
DIM = 1024
NC = 2
NS = 16
NW = NC * NS

EW = SEQ_LEN // NW
CR = 16
CH = CR * DIM
NCH = EW // CR
NSTEP = NCH * BATCH
NB = 6
LEAD = 3


def _row_off(base, k):
    c, b = k // BATCH, k % BATCH
    return b * SEQ_LEN + base + c * CR


def _sc_body(in_hbm, emb_hbm, out_hbm,
             emb_buf, io_buf, sem_emb, sem_in, sem_out):
    wid = lax.axis_index("s") * NC + lax.axis_index("c")
    base = wid * EW

    def in_cp(k):
        m = k % NB
        return pltpu.make_async_copy(
            in_hbm.at[pl.ds(_row_off(base, k), CR)], io_buf.at[m], sem_in.at[m])

    def out_cp(k):
        m = k % NB
        return pltpu.make_async_copy(
            io_buf.at[m], out_hbm.at[pl.ds(_row_off(base, k), CR)], sem_out.at[m])

    def emb_cp(c):
        return pltpu.make_async_copy(
            emb_hbm.at[pl.ds(base + c * CR, CR)], emb_buf, sem_emb)

    emb_cp(0).start()
    for j in range(LEAD):
        in_cp(j).start()

    for k in range(NSTEP):
        c, b = k // BATCH, k % BATCH
        m = k % NB
        if b == 0:
            emb_cp(c).wait()
        in_cp(k).wait()

        @plsc.parallel_loop(0, CH, step=16, unroll=8)
        def _add(i):
            r = i >> 10
            j2 = pl.multiple_of(i & (DIM - 1), 16)
            io_buf[m, r, pl.ds(j2, 16)] += emb_buf[r, pl.ds(j2, 16)]

        if b == BATCH - 1 and c + 1 < NCH:
            emb_cp(c + 1).start()
        out_cp(k).start()
        if k + LEAD < NSTEP:
            if k + LEAD - NB >= 0:
                out_cp(k + LEAD - NB).wait()
            in_cp(k + LEAD).start()

    for j in range(NSTEP - NB, NSTEP):
        out_cp(j).wait()


def kernel(inputs, embeddings):
    in2d = inputs.reshape(BATCH * SEQ_LEN, DIM)
    mesh = plsc.VectorSubcoreMesh(core_axis_name="c", subcore_axis_name="s")
    out = pl.kernel(
        _sc_body,
        out_type=jax.ShapeDtypeStruct((BATCH * SEQ_LEN, DIM), jnp.float32),
        mesh=mesh,
        scratch_types=[
            pltpu.VMEM((CR, DIM), jnp.float32),
            pltpu.VMEM((NB, CR, DIM), jnp.float32),
            pltpu.SemaphoreType.DMA,
            pltpu.SemaphoreType.DMA((NB,)),
            pltpu.SemaphoreType.DMA((NB,)),
        ],
    )(in2d, embeddings)
    return out.reshape(BATCH, SEQ_LEN, DIM)


# speedup vs baseline: 1.1351x; 1.1351x over previous
import jax
import jax.numpy as jnp
from jax import lax
from jax.experimental import pallas as pl
from jax.experimental.pallas import tpu as pltpu
from jax.experimental.pallas import tpu_sc as plsc

BATCH = 4
SEQ_LEN = 8192
DIM = 1024
NC = 2
NS = 16
NW = NC * NS

EW = SEQ_LEN // NW
CR = 16
CH = CR * DIM
NCH = EW // CR
NSTEP = NCH * BATCH
NB = 5


def _row_off(base, k):
    c, b = k // BATCH, k % BATCH
    return b * SEQ_LEN + base + c * CR


def _sc_body(in_hbm, emb_hbm, out_hbm,
             emb_buf, io_buf, sem_emb, sem_in, sem_out):
    wid = lax.axis_index("s") * NC + lax.axis_index("c")
    base = wid * EW

    def in_cp(k):
        m = k % NB
        return pltpu.make_async_copy(
            in_hbm.at[pl.ds(_row_off(base, k), CR)], io_buf.at[m], sem_in.at[m])

    def out_cp(k):
        m = k % NB
        return pltpu.make_async_copy(
            io_buf.at[m], out_hbm.at[pl.ds(_row_off(base, k), CR)], sem_out.at[m])

    def emb_cp(c):
        q = c % 2
        return pltpu.make_async_copy(
            emb_hbm.at[pl.ds(base + c * CR, CR)], emb_buf.at[q], sem_emb.at[q])

    emb_cp(0).start()
    in_cp(0).start()
    in_cp(1).start()
    in_cp(2).start()

    for k in range(NSTEP):
        m = k % NB
        c, b = k // BATCH, k % BATCH
        q = c % 2
        if b == 0:
            emb_cp(c).wait()
            if c + 1 < NCH:
                emb_cp(c + 1).start()
        in_cp(k).wait()
        if k + 3 < NSTEP:
            if k >= 2:
                out_cp(k - 2).wait()
            in_cp(k + 3).start()

        @plsc.parallel_loop(0, CH, step=16, unroll=8)
        def _add(i):
            r = i >> 10
            j = pl.multiple_of(i & (DIM - 1), 16)
            io_buf[m, r, pl.ds(j, 16)] += emb_buf[q, r, pl.ds(j, 16)]

        out_cp(k).start()

    out_cp(NSTEP - 3).wait()
    out_cp(NSTEP - 2).wait()
    out_cp(NSTEP - 1).wait()


def kernel(inputs, embeddings):
    in2d = inputs.reshape(BATCH * SEQ_LEN, DIM)
    mesh = plsc.VectorSubcoreMesh(core_axis_name="c", subcore_axis_name="s")
    out = pl.kernel(
        _sc_body,
        out_type=jax.ShapeDtypeStruct((BATCH * SEQ_LEN, DIM), jnp.float32),
        mesh=mesh,
        scratch_types=[
            pltpu.VMEM((2, CR, DIM), jnp.float32),
            pltpu.VMEM((NB, CR, DIM), jnp.float32),
            pltpu.SemaphoreType.DMA((2,)),
            pltpu.SemaphoreType.DMA((NB,)),
            pltpu.SemaphoreType.DMA((NB,)),
        ],
    )(in2d, embeddings)
    return out.reshape(BATCH, SEQ_LEN, DIM)
